# baseline (speedup 1.0000x reference)
import jax
import jax.numpy as jnp
from jax import lax
from jax.experimental import pallas as pl
from jax.experimental.pallas import tpu as pltpu

_B, _S, _D, _OUT, _E, _K = 1, 2048, 768, 768, 8, 2
_TM = 1024


def _moe_body(x_ref, wg1_ref, wg2_ref, w_ref, b_ref, y_ref):
    x = x_ref[...]
    h = jnp.tanh(
        lax.dot_general(x, wg1_ref[...], (((1,), (1,)), ((), ())),
                        preferred_element_type=jnp.float32))
    logits = lax.dot_general(h, wg2_ref[...], (((1,), (1,)), ((), ())),
                             preferred_element_type=jnp.float32)
    m1 = jnp.max(logits, axis=1, keepdims=True)
    col = lax.broadcasted_iota(jnp.int32, (_TM, _E), 1)
    i1 = jnp.argmax(logits, axis=1)[:, None]
    masked = jnp.where(col == i1, -jnp.inf, logits)
    m2 = jnp.max(masked, axis=1, keepdims=True)
    i2 = jnp.argmax(masked, axis=1)[:, None]
    s1 = 1.0 / (1.0 + jnp.exp(m2 - m1))
    s2 = 1.0 - s1
    combine = (jnp.where(col == i1, s1, 0.0)
               + jnp.where(col == i2, s2, 0.0))
    acc = lax.dot_general(combine, b_ref[...], (((1,), (0,)), ((), ())),
                          preferred_element_type=jnp.float32)
    for e in range(_E):
        ye = lax.dot_general(x, w_ref[e], (((1,), (1,)), ((), ())),
                             preferred_element_type=jnp.float32)
        acc = acc + combine[:, e:e + 1] * ye
    y_ref[...] = acc


def kernel(x, Wg1, Wg2, W, b):
    bs, sl, d = x.shape
    xf = x.reshape(-1, d)
    T = xf.shape[0]
    y = pl.pallas_call(
        _moe_body,
        grid=(T // _TM,),
        in_specs=[
            pl.BlockSpec((_TM, _D), lambda i: (i, 0)),
            pl.BlockSpec((_E, _D), lambda i: (0, 0)),
            pl.BlockSpec((_E, _E), lambda i: (0, 0)),
            pl.BlockSpec((_E, _OUT, _D), lambda i: (0, 0, 0)),
            pl.BlockSpec((_E, _OUT), lambda i: (0, 0)),
        ],
        out_specs=pl.BlockSpec((_TM, _OUT), lambda i: (i, 0)),
        out_shape=jax.ShapeDtypeStruct((T, _OUT), jnp.float32),
    )(xf, Wg1, Wg2, W, b)
    return y.reshape(bs, sl, _OUT), jnp.float32(-100.0)


# speedup vs baseline: 1.0048x; 1.0048x over previous
import jax
import jax.numpy as jnp
from jax import lax
from jax.experimental import pallas as pl
from jax.experimental.pallas import tpu as pltpu

_B, _S, _D, _OUT, _E, _K = 1, 2048, 768, 768, 8, 2
_TM = 512


def _moe_body(x_ref, wg1_ref, wg2_ref, w_ref, b_ref, y_ref):
    x = x_ref[...]
    h = jnp.tanh(
        lax.dot_general(x, wg1_ref[...], (((1,), (1,)), ((), ())),
                        preferred_element_type=jnp.float32))
    logits = lax.dot_general(h, wg2_ref[...], (((1,), (1,)), ((), ())),
                             preferred_element_type=jnp.float32)
    m1 = jnp.max(logits, axis=1, keepdims=True)
    col = lax.broadcasted_iota(jnp.int32, (_TM, _E), 1)
    i1 = jnp.argmax(logits, axis=1)[:, None]
    masked = jnp.where(col == i1, -jnp.inf, logits)
    m2 = jnp.max(masked, axis=1, keepdims=True)
    i2 = jnp.argmax(masked, axis=1)[:, None]
    s1 = 1.0 / (1.0 + jnp.exp(m2 - m1))
    s2 = 1.0 - s1
    combine = (jnp.where(col == i1, s1, 0.0)
               + jnp.where(col == i2, s2, 0.0))
    acc = lax.dot_general(combine, b_ref[...], (((1,), (0,)), ((), ())),
                          preferred_element_type=jnp.float32)
    for e in range(_E):
        ye = lax.dot_general(x, w_ref[e], (((1,), (1,)), ((), ())),
                             preferred_element_type=jnp.float32)
        acc = acc + combine[:, e:e + 1] * ye
    y_ref[...] = acc


def kernel(x, Wg1, Wg2, W, b):
    bs, sl, d = x.shape
    xf = x.reshape(-1, d)
    T = xf.shape[0]
    y = pl.pallas_call(
        _moe_body,
        grid=(T // _TM,),
        in_specs=[
            pl.BlockSpec((_TM, _D), lambda i: (i, 0)),
            pl.BlockSpec((_E, _D), lambda i: (0, 0)),
            pl.BlockSpec((_E, _E), lambda i: (0, 0)),
            pl.BlockSpec((_E, _OUT, _D), lambda i: (0, 0, 0)),
            pl.BlockSpec((_E, _OUT), lambda i: (0, 0)),
        ],
        out_specs=pl.BlockSpec((_TM, _OUT), lambda i: (i, 0)),
        out_shape=jax.ShapeDtypeStruct((T, _OUT), jnp.float32),
    )(xf, Wg1, Wg2, W, b)
    return y.reshape(bs, sl, _OUT), jnp.float32(-100.0)
